# baseline (speedup 1.0000x reference)
import functools

import jax
import jax.numpy as jnp
from jax import lax
from jax.experimental import pallas as pl
from jax.experimental.pallas import tpu as pltpu
from jax.experimental.pallas import tpu_sc as plsc

D = 128
V = 257
L = 1024
P_ROWS = 2 * L - 1
FILL = L - 129
NC = 2
NS = 16
FB = 128
NDMA = 8
RPD = 46
RPS = 36
W = 512
NCHUNK = 5
NBUF = 2


def _sc_body(table_hbm, out_hbm, p_sh, fill_v, trow_v, span_v, idx_v,
             sem_g, sem_o):
    c = lax.axis_index("c")
    s = lax.axis_index("s")
    p = (s - NDMA) // 2
    g = (s - NDMA) % 2
    i0s = c * 512 + NDMA * RPD + RPS * p
    span0 = L - i0s - RPS + W * g

    gathers = [
        pltpu.make_async_copy(table_hbm.at[idx_v.at[j]],
                              span_v.at[pl.ds(j * 128, 128)], sem_g)
        for j in range(NCHUNK)
    ]

    @pl.when(s >= NDMA)
    def _():
        lanes = lax.iota(jnp.int32, 16)

        def fill_idx(j, carry):
            for k in range(8):
                base = span0 - FILL + j * 128 + k * 16
                idx_v[j, pl.ds(k * 16, 16)] = jnp.clip(base + lanes, 0, V - 1)
            return carry

        lax.fori_loop(0, NCHUNK, fill_idx, 0)
        for cp in gathers:
            cp.start()

    @pl.when(s == 0)
    def _():
        pltpu.sync_copy(table_hbm, p_sh.at[pl.ds(FILL, V)])

    def _replicate(edge_row):
        pltpu.sync_copy(table_hbm.at[pl.ds(edge_row, 1)], trow_v)
        row = [trow_v[0, pl.ds(k * 16, 16)] for k in range(D // 16)]

        def rep(r, carry):
            for k in range(D // 16):
                fill_v[r, pl.ds(k * 16, 16)] = row[k]
            return carry

        lax.fori_loop(0, FB, rep, 0)

    for b in range(7):
        size = FILL % FB if b == 0 else FB
        off = 0 if b == 0 else FILL % FB + (b - 1) * FB

        @pl.when(s == 1 + b)
        def _(size=size, off=off):
            _replicate(0)
            pltpu.sync_copy(fill_v.at[pl.ds(0, size)],
                            p_sh.at[pl.ds(off, size)])

    for b in range(7):
        size = FB if b < 6 else FILL % FB
        off = FILL + V + b * FB

        @pl.when(s == 8 + b)
        def _(size=size, off=off):
            _replicate(V - 1)
            pltpu.sync_copy(fill_v.at[pl.ds(0, size)],
                            p_sh.at[pl.ds(off, size)])

    @pl.when(s >= NDMA)
    def _():
        for cp in gathers:
            cp.wait()

    plsc.subcore_barrier()

    @pl.when(s < NDMA)
    def _():
        i0 = c * 512 + RPD * s
        inflight = []
        for k in range(RPD):
            i = i0 + k
            if len(inflight) >= NBUF:
                inflight.pop(0).wait()
            inflight.append(
                pltpu.async_copy(p_sh.at[pl.ds(L - 1 - i, L)],
                                 out_hbm.at[i], sem_o))
        for cp in inflight:
            cp.wait()

    @pl.when(s >= NDMA)
    def _():
        inflight = []
        for q in range(RPS):
            if len(inflight) >= 2 * NBUF:
                inflight.pop(0).wait()
            inflight.append(
                pltpu.async_copy(span_v.at[pl.ds(RPS - 1 - q, W)],
                                 out_hbm.at[i0s + q, pl.ds(W * g, W)], sem_o))
        for cp in inflight:
            cp.wait()


@functools.partial(
    pl.kernel,
    out_type=jax.ShapeDtypeStruct((L, L, D), jnp.float32),
    mesh=plsc.VectorSubcoreMesh(core_axis_name="c", subcore_axis_name="s"),
    scratch_types=[
        pltpu.VMEM_SHARED((P_ROWS, D), jnp.float32),
        pltpu.VMEM((FB, D), jnp.float32),
        pltpu.VMEM((1, D), jnp.float32),
        pltpu.VMEM((NCHUNK * 128, D), jnp.float32),
        pltpu.VMEM((NCHUNK, 128), jnp.int32),
        pltpu.SemaphoreType.DMA,
        pltpu.SemaphoreType.DMA,
    ],
)
def _rel_pos_sc(table_hbm, out_hbm, p_sh, fill_v, trow_v, span_v, idx_v,
                sem_g, sem_o):
    _sc_body(table_hbm, out_hbm, p_sh, fill_v, trow_v, span_v, idx_v,
             sem_g, sem_o)


def kernel(embeddings_table, length):
    return _rel_pos_sc(embeddings_table)


# speedup vs baseline: 2.4920x; 2.4920x over previous
import functools

import jax
import jax.numpy as jnp
from jax import lax
from jax.experimental import pallas as pl
from jax.experimental.pallas import tpu as pltpu
from jax.experimental.pallas import tpu_sc as plsc

D = 128
V = 257
L = 1024
P_ROWS = 2 * L - 1
FILL = L - 129
NC = 2
NS = 16
ROWS_PER_TILE = L // (NC * NS)
FB = 128
NDMA = 8
RPD = 46
RPS = 36
W = 512
NBUF = 2


def _sc_body(table_hbm, out_hbm, p_sh, fill_v, trow_v, span_v, sem_o):
    c = lax.axis_index("c")
    s = lax.axis_index("s")

    @pl.when(s == 0)
    def _():
        pltpu.sync_copy(table_hbm, p_sh.at[pl.ds(FILL, V)])

    def _build_fill(edge_row):
        pltpu.sync_copy(table_hbm.at[pl.ds(edge_row, 1)], trow_v)

        def rep(r, carry):
            for k in range(D // 16):
                fill_v[r, pl.ds(k * 16, 16)] = trow_v[0, pl.ds(k * 16, 16)]
            return carry

        lax.fori_loop(0, FB, rep, 0)

    @pl.when(s == 1)
    def _():
        _build_fill(0)
        pltpu.sync_copy(fill_v.at[pl.ds(0, FILL % FB)],
                        p_sh.at[pl.ds(0, FILL % FB)])
        for b in range(FILL // FB):
            pltpu.sync_copy(fill_v, p_sh.at[pl.ds(FILL % FB + b * FB, FB)])

    @pl.when(s == 2)
    def _():
        _build_fill(V - 1)
        for b in range(FILL // FB):
            pltpu.sync_copy(fill_v, p_sh.at[pl.ds(FILL + V + b * FB, FB)])
        pltpu.sync_copy(fill_v.at[pl.ds(0, FILL % FB)],
                        p_sh.at[pl.ds(P_ROWS - FILL % FB, FILL % FB)])

    plsc.subcore_barrier()

    @pl.when(s < NDMA)
    def _():
        i0 = c * 512 + RPD * s
        inflight = []
        for k in range(RPD):
            i = i0 + k
            if len(inflight) >= NBUF:
                inflight.pop(0).wait()
            inflight.append(
                pltpu.async_copy(p_sh.at[pl.ds(L - 1 - i, L)],
                                 out_hbm.at[i], sem_o))
        for cp in inflight:
            cp.wait()

    @pl.when(s >= NDMA)
    def _():
        p = (s - NDMA) // 2
        g = (s - NDMA) % 2
        i0 = c * 512 + NDMA * RPD + RPS * p
        span0 = L - i0 - RPS + W * g
        pltpu.sync_copy(p_sh.at[pl.ds(span0, RPS + W)], span_v)
        inflight = []
        for q in range(RPS):
            if len(inflight) >= 2 * NBUF:
                inflight.pop(0).wait()
            inflight.append(
                pltpu.async_copy(span_v.at[pl.ds(RPS - 1 - q, W)],
                                 out_hbm.at[i0 + q, pl.ds(W * g, W)], sem_o))
        for cp in inflight:
            cp.wait()


@functools.partial(
    pl.kernel,
    out_type=jax.ShapeDtypeStruct((L, L, D), jnp.float32),
    mesh=plsc.VectorSubcoreMesh(core_axis_name="c", subcore_axis_name="s"),
    scratch_types=[
        pltpu.VMEM_SHARED((P_ROWS, D), jnp.float32),
        pltpu.VMEM((FB, D), jnp.float32),
        pltpu.VMEM((1, D), jnp.float32),
        pltpu.VMEM((RPS + W, D), jnp.float32),
        pltpu.SemaphoreType.DMA,
    ],
)
def _rel_pos_sc(table_hbm, out_hbm, p_sh, fill_v, trow_v, span_v, sem_o):
    _sc_body(table_hbm, out_hbm, p_sh, fill_v, trow_v, span_v, sem_o)


def kernel(embeddings_table, length):
    return _rel_pos_sc(embeddings_table)


# speedup vs baseline: 2.4934x; 1.0005x over previous
import functools

import jax
import jax.numpy as jnp
from jax import lax
from jax.experimental import pallas as pl
from jax.experimental.pallas import tpu as pltpu
from jax.experimental.pallas import tpu_sc as plsc

D = 128
V = 257
L = 1024
P_ROWS = 2 * L - 1
FILL = L - 129
NC = 2
NS = 16
ROWS_PER_TILE = L // (NC * NS)
FB = 128
NDMA = 8
RPD = 46
RPS = 36
W = 512
NBUF = 3


def _sc_body(table_hbm, out_hbm, p_sh, fill_v, trow_v, span_v, sem_o):
    c = lax.axis_index("c")
    s = lax.axis_index("s")

    @pl.when(s == 0)
    def _():
        pltpu.sync_copy(table_hbm, p_sh.at[pl.ds(FILL, V)])

    def _build_fill(edge_row):
        pltpu.sync_copy(table_hbm.at[pl.ds(edge_row, 1)], trow_v)

        def rep(r, carry):
            for k in range(D // 16):
                fill_v[r, pl.ds(k * 16, 16)] = trow_v[0, pl.ds(k * 16, 16)]
            return carry

        lax.fori_loop(0, FB, rep, 0)

    @pl.when(s == 1)
    def _():
        _build_fill(0)
        pltpu.sync_copy(fill_v.at[pl.ds(0, FILL % FB)],
                        p_sh.at[pl.ds(0, FILL % FB)])
        for b in range(FILL // FB):
            pltpu.sync_copy(fill_v, p_sh.at[pl.ds(FILL % FB + b * FB, FB)])

    @pl.when(s == 2)
    def _():
        _build_fill(V - 1)
        for b in range(FILL // FB):
            pltpu.sync_copy(fill_v, p_sh.at[pl.ds(FILL + V + b * FB, FB)])
        pltpu.sync_copy(fill_v.at[pl.ds(0, FILL % FB)],
                        p_sh.at[pl.ds(P_ROWS - FILL % FB, FILL % FB)])

    plsc.subcore_barrier()

    @pl.when(s < NDMA)
    def _():
        i0 = c * 512 + RPD * s
        inflight = []
        for k in range(RPD):
            i = i0 + k
            if len(inflight) >= NBUF:
                inflight.pop(0).wait()
            inflight.append(
                pltpu.async_copy(p_sh.at[pl.ds(L - 1 - i, L)],
                                 out_hbm.at[i], sem_o))
        for cp in inflight:
            cp.wait()

    @pl.when(s >= NDMA)
    def _():
        p = (s - NDMA) // 2
        g = (s - NDMA) % 2
        i0 = c * 512 + NDMA * RPD + RPS * p
        span0 = L - i0 - RPS + W * g
        pltpu.sync_copy(p_sh.at[pl.ds(span0, RPS + W)], span_v)
        inflight = []
        for q in range(RPS):
            if len(inflight) >= 2 * NBUF:
                inflight.pop(0).wait()
            inflight.append(
                pltpu.async_copy(span_v.at[pl.ds(RPS - 1 - q, W)],
                                 out_hbm.at[i0 + q, pl.ds(W * g, W)], sem_o))
        for cp in inflight:
            cp.wait()


@functools.partial(
    pl.kernel,
    out_type=jax.ShapeDtypeStruct((L, L, D), jnp.float32),
    mesh=plsc.VectorSubcoreMesh(core_axis_name="c", subcore_axis_name="s"),
    scratch_types=[
        pltpu.VMEM_SHARED((P_ROWS, D), jnp.float32),
        pltpu.VMEM((FB, D), jnp.float32),
        pltpu.VMEM((1, D), jnp.float32),
        pltpu.VMEM((RPS + W, D), jnp.float32),
        pltpu.SemaphoreType.DMA,
    ],
)
def _rel_pos_sc(table_hbm, out_hbm, p_sh, fill_v, trow_v, span_v, sem_o):
    _sc_body(table_hbm, out_hbm, p_sh, fill_v, trow_v, span_v, sem_o)


def kernel(embeddings_table, length):
    return _rel_pos_sc(embeddings_table)
